# initial kernel scaffold (unmeasured)
import jax
import jax.numpy as jnp
from jax import lax
from jax.experimental import pallas as pl
from jax.experimental.pallas import tpu as pltpu


def kernel(
    x,
):
    def body(*refs):
        pass

    out_shape = jax.ShapeDtypeStruct(..., jnp.float32)
    return pl.pallas_call(body, out_shape=out_shape)(...)


# baseline (device time: 1098178 ns/iter reference)
import jax
import jax.numpy as jnp
from jax import lax
from jax.experimental import pallas as pl
from jax.experimental.pallas import tpu as pltpu


def kernel(x):
    xb = x.astype(jnp.bfloat16)
    m, n = xb.shape

    def body(x_ref, out_ref, local_sem, send_sem, recv_sem):
        my_x = lax.axis_index("x")
        my_y = lax.axis_index("y")
        nbr = (my_x, 1 - my_y)

        barrier = pltpu.get_barrier_semaphore()
        pl.semaphore_signal(
            barrier, inc=1, device_id=nbr, device_id_type=pl.DeviceIdType.MESH
        )
        pl.semaphore_wait(barrier, 1)

        local = pltpu.make_async_copy(
            x_ref, out_ref.at[pl.ds(my_y * m, m)], local_sem
        )
        local.start()

        rdma = pltpu.make_async_remote_copy(
            src_ref=x_ref,
            dst_ref=out_ref.at[pl.ds(my_y * m, m)],
            send_sem=send_sem,
            recv_sem=recv_sem,
            device_id=nbr,
            device_id_type=pl.DeviceIdType.MESH,
        )
        rdma.start()

        local.wait()
        rdma.wait()

    return pl.pallas_call(
        body,
        out_shape=jax.ShapeDtypeStruct((2 * m, n), xb.dtype),
        in_specs=[pl.BlockSpec(memory_space=pltpu.MemorySpace.HBM)],
        out_specs=pl.BlockSpec(memory_space=pltpu.MemorySpace.HBM),
        scratch_shapes=[
            pltpu.SemaphoreType.DMA,
            pltpu.SemaphoreType.DMA,
            pltpu.SemaphoreType.DMA,
        ],
        compiler_params=pltpu.CompilerParams(collective_id=0),
    )(xb)


# device time: 255447 ns/iter; 4.2990x vs baseline; 4.2990x over previous
import functools

import jax
import jax.numpy as jnp
from jax import lax
from jax.experimental import pallas as pl
from jax.experimental.pallas import tpu as pltpu


def kernel(x):
    m, n = x.shape
    H = m // 2
    CH = 1024
    K = H // CH

    def body(x_ref, out_ref, vshard, vload, lsems, ssem,
             ysend, yrecv, xsend, xrecv):
        my_x = lax.axis_index("x")
        my_y = lax.axis_index("y")
        ynbr = (my_x, 1 - my_y)
        xnbr = (1 - my_x, my_y)

        bar = pltpu.get_barrier_semaphore()
        for nbr in (ynbr, xnbr):
            pl.semaphore_signal(bar, inc=1, device_id=nbr,
                                device_id_type=pl.DeviceIdType.MESH)
        pl.semaphore_wait(bar, 2)

        dhalf = my_x * H
        phalf = (1 - my_x) * H

        def lofs(i):
            if i < K:
                return dhalf + i * CH
            return phalf + (i - K) * CH

        def start_load(i):
            slot = i % 2
            c = pltpu.make_async_copy(
                x_ref.at[pl.ds(lofs(i), CH)], vload.at[slot], lsems.at[slot])
            c.start()
            return c

        loads = {0: start_load(0)}
        y_rdmas = []
        fw_rdmas = []
        for i in range(2 * K):
            if i + 1 < 2 * K:
                loads[i + 1] = start_load(i + 1)
            loads[i].wait()
            slot = i % 2
            vshard[pl.ds(lofs(i), CH), :] = vload[slot].astype(jnp.bfloat16)
            if i < K:
                r = pltpu.make_async_remote_copy(
                    src_ref=vshard.at[pl.ds(dhalf + i * CH, CH)],
                    dst_ref=out_ref.at[pl.ds(my_y * m + dhalf + i * CH, CH)],
                    send_sem=ysend.at[i], recv_sem=yrecv.at[i],
                    device_id=ynbr, device_id_type=pl.DeviceIdType.MESH)
                r.start()
                y_rdmas.append(r)
            else:
                j = i - K
                y_rdmas[j].wait_recv()
                rofs = (1 - my_y) * m + dhalf + j * CH
                fw = pltpu.make_async_remote_copy(
                    src_ref=out_ref.at[pl.ds(rofs, CH)],
                    dst_ref=out_ref.at[pl.ds(rofs, CH)],
                    send_sem=xsend.at[j], recv_sem=xrecv.at[j],
                    device_id=xnbr, device_id_type=pl.DeviceIdType.MESH)
                fw.start()
                fw_rdmas.append(fw)

        st = pltpu.make_async_copy(
            vshard, out_ref.at[pl.ds(my_y * m, m)], ssem)
        st.start()

        for r in y_rdmas:
            r.wait_send()
        for r in fw_rdmas:
            r.wait_send()
        for r in fw_rdmas:
            r.wait_recv()
        st.wait()

        @functools.partial(pl.run_scoped,
                           second_barrier=pltpu.SemaphoreType.REGULAR)
        def _(second_barrier):
            for nbr in (ynbr, xnbr):
                pl.semaphore_signal(second_barrier, inc=1, device_id=nbr,
                                    device_id_type=pl.DeviceIdType.MESH)
            pl.semaphore_wait(second_barrier, 2)

    return pl.pallas_call(
        body,
        out_shape=jax.ShapeDtypeStruct((2 * m, n), jnp.bfloat16),
        in_specs=[pl.BlockSpec(memory_space=pltpu.MemorySpace.HBM)],
        out_specs=pl.BlockSpec(memory_space=pltpu.MemorySpace.HBM),
        scratch_shapes=[
            pltpu.VMEM((m, n), jnp.bfloat16),
            pltpu.VMEM((2, CH, n), jnp.float32),
            pltpu.SemaphoreType.DMA((2,)),
            pltpu.SemaphoreType.DMA,
            pltpu.SemaphoreType.DMA((K,)),
            pltpu.SemaphoreType.DMA((K,)),
            pltpu.SemaphoreType.DMA((K,)),
            pltpu.SemaphoreType.DMA((K,)),
        ],
        compiler_params=pltpu.CompilerParams(
            collective_id=0, vmem_limit_bytes=48 * 1024 * 1024),
    )(x)


# device time: 244502 ns/iter; 4.4915x vs baseline; 1.0448x over previous
import functools

import jax
import jax.numpy as jnp
from jax import lax
from jax.experimental import pallas as pl
from jax.experimental.pallas import tpu as pltpu


def kernel(x):
    m, n = x.shape
    H = m // 2
    CH = 512
    K = H // CH

    def body(x_ref, out_ref, vshard, vload, lsems, ssem,
             ysend, yrecv, xsend, xrecv):
        my_x = lax.axis_index("x")
        my_y = lax.axis_index("y")
        ynbr = (my_x, 1 - my_y)
        xnbr = (1 - my_x, my_y)

        bar = pltpu.get_barrier_semaphore()
        for nbr in (ynbr, xnbr):
            pl.semaphore_signal(bar, inc=1, device_id=nbr,
                                device_id_type=pl.DeviceIdType.MESH)
        pl.semaphore_wait(bar, 2)

        dhalf = my_x * H
        phalf = (1 - my_x) * H

        def lofs(i):
            if i < K:
                return dhalf + i * CH
            return phalf + (i - K) * CH

        def start_load(i):
            slot = i % 2
            c = pltpu.make_async_copy(
                x_ref.at[pl.ds(lofs(i), CH)], vload.at[slot], lsems.at[slot])
            c.start()
            return c

        loads = {0: start_load(0)}
        y_rdmas = []
        fw_rdmas = []
        for i in range(2 * K):
            if i + 1 < 2 * K:
                loads[i + 1] = start_load(i + 1)
            loads[i].wait()
            slot = i % 2
            vshard[pl.ds(lofs(i), CH), :] = vload[slot].astype(jnp.bfloat16)
            if i < K:
                r = pltpu.make_async_remote_copy(
                    src_ref=vshard.at[pl.ds(dhalf + i * CH, CH)],
                    dst_ref=out_ref.at[pl.ds(my_y * m + dhalf + i * CH, CH)],
                    send_sem=ysend.at[i], recv_sem=yrecv.at[i],
                    device_id=ynbr, device_id_type=pl.DeviceIdType.MESH)
                r.start()
                y_rdmas.append(r)
            else:
                j = i - K
                y_rdmas[j].wait_recv()
                rofs = (1 - my_y) * m + dhalf + j * CH
                fw = pltpu.make_async_remote_copy(
                    src_ref=out_ref.at[pl.ds(rofs, CH)],
                    dst_ref=out_ref.at[pl.ds(rofs, CH)],
                    send_sem=xsend.at[j], recv_sem=xrecv.at[j],
                    device_id=xnbr, device_id_type=pl.DeviceIdType.MESH)
                fw.start()
                fw_rdmas.append(fw)

        st = pltpu.make_async_copy(
            vshard, out_ref.at[pl.ds(my_y * m, m)], ssem)
        st.start()

        for r in y_rdmas:
            r.wait_send()
        for r in fw_rdmas:
            r.wait_send()
        for r in fw_rdmas:
            r.wait_recv()
        st.wait()

        @functools.partial(pl.run_scoped,
                           second_barrier=pltpu.SemaphoreType.REGULAR)
        def _(second_barrier):
            for nbr in (ynbr, xnbr):
                pl.semaphore_signal(second_barrier, inc=1, device_id=nbr,
                                    device_id_type=pl.DeviceIdType.MESH)
            pl.semaphore_wait(second_barrier, 2)

    return pl.pallas_call(
        body,
        out_shape=jax.ShapeDtypeStruct((2 * m, n), jnp.bfloat16),
        in_specs=[pl.BlockSpec(memory_space=pltpu.MemorySpace.HBM)],
        out_specs=pl.BlockSpec(memory_space=pltpu.MemorySpace.HBM),
        scratch_shapes=[
            pltpu.VMEM((m, n), jnp.bfloat16),
            pltpu.VMEM((2, CH, n), jnp.float32),
            pltpu.SemaphoreType.DMA((2,)),
            pltpu.SemaphoreType.DMA,
            pltpu.SemaphoreType.DMA((K,)),
            pltpu.SemaphoreType.DMA((K,)),
            pltpu.SemaphoreType.DMA((K,)),
            pltpu.SemaphoreType.DMA((K,)),
        ],
        compiler_params=pltpu.CompilerParams(
            collective_id=0, vmem_limit_bytes=48 * 1024 * 1024),
    )(x)
